# baseline (speedup 1.0000x reference)
import functools

import jax
import jax.numpy as jnp
from jax import lax
from jax.experimental import pallas as pl
from jax.experimental.pallas import tpu as pltpu
from jax.experimental.pallas import tpu_sc as plsc

_BATCH = 16384
_K = 32
_NC = 2
_NS = 16
_NW = _NC * _NS
_BPW = _BATCH // _NW
_CHUNK = 128
_NCHUNK = _BPW // _CHUNK
_LANES = 16


def _mf_body(user_hbm, movie_hbm, w_hbm, u_hbm, b_hbm, c_hbm, out_hbm,
             idx_u, idx_m, w_v, u_v, b_v, c_v, out_v, sem):
    wid = lax.axis_index("s") * _NC + lax.axis_index("c")
    base = wid * _BPW

    for j in range(_NCHUNK):
        pltpu.sync_copy(user_hbm.at[pl.ds(base + j * _CHUNK, _CHUNK)],
                        idx_u.at[j])
        pltpu.sync_copy(movie_hbm.at[pl.ds(base + j * _CHUNK, _CHUNK)],
                        idx_m.at[j])

    copies = []
    for j in range(_NCHUNK):
        copies.append(pltpu.async_copy(w_hbm.at[idx_u.at[j]], w_v.at[j], sem))
        copies.append(pltpu.async_copy(u_hbm.at[idx_m.at[j]], u_v.at[j], sem))
        copies.append(pltpu.async_copy(b_hbm.at[idx_u.at[j]], b_v.at[j], sem))
        copies.append(pltpu.async_copy(c_hbm.at[idx_m.at[j]], c_v.at[j], sem))
    for c in copies:
        c.wait()

    iota = lax.iota(jnp.int32, _LANES)

    def chunk_body(ci, _):
        j = ci // (_CHUNK // _LANES)
        rows = (ci % (_CHUNK // _LANES)) * _LANES + iota
        jv = jnp.full((_LANES,), j, jnp.int32)
        acc = (plsc.load_gather(b_v, [jv, rows])
               + plsc.load_gather(c_v, [jv, rows]))
        for k in range(_K):
            kv = jnp.full((_LANES,), k, jnp.int32)
            wv = plsc.load_gather(w_v, [jv, rows, kv])
            uv = plsc.load_gather(u_v, [jv, rows, kv])
            acc = acc + wv * uv
        out_v[pl.ds(ci * _LANES, _LANES)] = acc
        return 0

    lax.fori_loop(0, _BPW // _LANES, chunk_body, 0)

    pltpu.sync_copy(out_v, out_hbm.at[pl.ds(base, _BPW)])


_mf_call = pl.kernel(
    _mf_body,
    out_type=jax.ShapeDtypeStruct((_BATCH,), jnp.float32),
    mesh=plsc.VectorSubcoreMesh(core_axis_name="c", subcore_axis_name="s"),
    compiler_params=pltpu.CompilerParams(needs_layout_passes=False,
                                         use_tc_tiling_on_sc=False),
    scratch_types=[
        pltpu.VMEM((_NCHUNK, _CHUNK), jnp.int32),
        pltpu.VMEM((_NCHUNK, _CHUNK), jnp.int32),
        pltpu.VMEM((_NCHUNK, _CHUNK, _K), jnp.float32),
        pltpu.VMEM((_NCHUNK, _CHUNK, _K), jnp.float32),
        pltpu.VMEM((_NCHUNK, _CHUNK), jnp.float32),
        pltpu.VMEM((_NCHUNK, _CHUNK), jnp.float32),
        pltpu.VMEM((_BPW,), jnp.float32),
        pltpu.SemaphoreType.DMA,
    ],
)


@jax.jit
def kernel(user, movie, W, U, B, C):
    return _mf_call(user.astype(jnp.int32), movie.astype(jnp.int32),
                    W, U, B.reshape(-1), C.reshape(-1))
